# Initial kernel scaffold; baseline (speedup 1.0000x reference)
import jax
import jax.numpy as jnp
from jax.experimental import pallas as pl


def kernel(seq, seq_table, pos_table):
    raise NotImplementedError("write your pallas kernel here")


# speedup vs baseline: 8.2968x; 8.2968x over previous
import functools

import jax
import jax.numpy as jnp
from jax import lax
from jax.experimental import pallas as pl
from jax.experimental.pallas import tpu as pltpu
from jax.experimental.pallas import tpu_sc as plsc

_BATCH = 4096
_MAX_LEN = 200
_EMBED = 64
_VOCAB = 32

_NROWS = _BATCH * _MAX_LEN
_FUSED_ROWS = _MAX_LEN * _VOCAB

_LANES = 16
_IDX_W = 128
_GPB = 4
_CHUNK = _GPB * _IDX_W


def _fused_body(tab_ref, pos_ref, out_ref):
    pos = pos_ref[...]
    tab = tab_ref[...]
    out_ref[...] = pos[:, None, :] + tab[None, :, :]


def _build_fused(seq_table, pos_table):
    out3 = pl.pallas_call(
        _fused_body,
        out_shape=jax.ShapeDtypeStruct((_MAX_LEN, _VOCAB, _EMBED), jnp.float32),
    )(seq_table, pos_table)
    return out3.reshape(_FUSED_ROWS, _EMBED)


def _sc_gather_body(nc, rows_per_tile, seq2d, fused, out,
                    idx_v, rows_v, sg0, sg1, ss0, ss1):
    sem_g = (sg0, sg1)
    sem_s = (ss0, ss1)
    wid = lax.axis_index("s") * nc + lax.axis_index("c")
    idx_rows = rows_per_tile // _IDX_W
    n_chunks = rows_per_tile // _CHUNK
    base = wid * idx_rows
    obase = wid * rows_per_tile

    pltpu.sync_copy(seq2d.at[pl.ds(base, idx_rows)], idx_v)
    iota = lax.iota(jnp.int32, _LANES)

    def idx_body(j, carry):
        for k in range(_IDX_W // _LANES):
            q = iota + (j * _IDX_W + k * _LANES)
            t = lax.rem(q, _MAX_LEN)
            sl = pl.ds(k * _LANES, _LANES)
            idx_v[j, sl] = idx_v[j, sl] + t * _VOCAB
        return carry

    lax.fori_loop(0, idx_rows, idx_body, 0)

    def fire(chunk, b):
        for g in range(_GPB):
            pltpu.async_copy(
                fused.at[idx_v.at[chunk * _GPB + g]],
                rows_v.at[b].at[pl.ds(g * _IDX_W, _IDX_W)],
                sem_g[b])

    def drain_gather(b):
        pltpu.make_async_copy(out.at[pl.ds(0, _CHUNK)], rows_v.at[b],
                              sem_g[b]).wait()

    def drain_store(b):
        pltpu.make_async_copy(out.at[pl.ds(0, _CHUNK)], rows_v.at[b],
                              sem_s[b]).wait()

    fire(0, 0)

    def chunk_body(i, carry):
        for b in range(2):
            chunk = i * 2 + b
            other = 1 - b

            @pl.when(chunk >= 1)
            def _():
                drain_store(other)

            @pl.when(chunk + 1 < n_chunks)
            def _():
                fire(chunk + 1, other)

            drain_gather(b)
            pltpu.async_copy(rows_v.at[b],
                             out.at[pl.ds(obase + chunk * _CHUNK, _CHUNK)],
                             sem_s[b])
        return carry

    lax.fori_loop(0, n_chunks // 2, chunk_body, 0)
    drain_store((n_chunks - 1) % 2)


def _sc_gather(seq2d, fused):
    info = plsc.get_sparse_core_info()
    nc, ns = info.num_cores, info.num_subcores
    nw = nc * ns
    rows_per_tile = _NROWS // nw
    mesh = plsc.VectorSubcoreMesh(core_axis_name="c", subcore_axis_name="s")
    kern = pl.kernel(
        functools.partial(_sc_gather_body, nc, rows_per_tile),
        out_type=jax.ShapeDtypeStruct((_NROWS, _EMBED), jnp.float32),
        mesh=mesh,
        compiler_params=pltpu.CompilerParams(use_tc_tiling_on_sc=False),
        scratch_types=[
            pltpu.VMEM((rows_per_tile // _IDX_W, _IDX_W), jnp.int32),
            pltpu.VMEM((2, _CHUNK, _EMBED), jnp.float32),
            pltpu.SemaphoreType.DMA,
            pltpu.SemaphoreType.DMA,
            pltpu.SemaphoreType.DMA,
            pltpu.SemaphoreType.DMA,
        ],
    )
    return kern(seq2d, fused)


def kernel(seq, seq_table, pos_table):
    seq2d = seq.astype(jnp.int32).reshape(_NROWS // _IDX_W, _IDX_W)
    fused = _build_fused(seq_table.astype(jnp.float32),
                         pos_table.astype(jnp.float32))
    out = _sc_gather(seq2d, fused)
    return out.reshape(_BATCH, _MAX_LEN, _EMBED)


# speedup vs baseline: 9.5061x; 1.1458x over previous
import functools

import jax
import jax.numpy as jnp
from jax import lax
from jax.experimental import pallas as pl
from jax.experimental.pallas import tpu as pltpu
from jax.experimental.pallas import tpu_sc as plsc

_BATCH = 4096
_MAX_LEN = 200
_EMBED = 64
_VOCAB = 32

_NROWS = _BATCH * _MAX_LEN
_FUSED_ROWS = _MAX_LEN * _VOCAB

_LANES = 16
_IDX_W = 128
_GPB = 4
_CHUNK = _GPB * _IDX_W


def _fused_body(tab_ref, pos_ref, out_ref):
    pos = pos_ref[...]
    tab = tab_ref[...]
    out_ref[...] = pos[:, None, :] + tab[None, :, :]


def _build_fused(seq_table, pos_table):
    out3 = pl.pallas_call(
        _fused_body,
        out_shape=jax.ShapeDtypeStruct((_MAX_LEN, _VOCAB, _EMBED), jnp.float32),
    )(seq_table, pos_table)
    return out3.reshape(_FUSED_ROWS, _EMBED)


def _sc_gather_body(nc, ns, rows_per_tile, seq2d, fused, out,
                    idx_v, rows_v, fused_sh, sg0, sg1, ss0, ss1):
    sem_g = (sg0, sg1)
    sem_s = (ss0, ss1)
    sid = lax.axis_index("s")
    wid = sid * nc + lax.axis_index("c")
    idx_rows = rows_per_tile // _IDX_W
    n_chunks = rows_per_tile // _CHUNK
    base = wid * idx_rows
    obase = wid * rows_per_tile

    fshard = _FUSED_ROWS // ns
    pltpu.sync_copy(fused.at[pl.ds(sid * fshard, fshard)],
                    fused_sh.at[pl.ds(sid * fshard, fshard)])

    pltpu.sync_copy(seq2d.at[pl.ds(base, idx_rows)], idx_v)
    iota = lax.iota(jnp.int32, _LANES)

    def idx_body(j, carry):
        for k in range(_IDX_W // _LANES):
            q = iota + (j * _IDX_W + k * _LANES)
            t = lax.rem(q, _MAX_LEN)
            sl = pl.ds(k * _LANES, _LANES)
            idx_v[j, sl] = idx_v[j, sl] + t * _VOCAB
        return carry

    lax.fori_loop(0, idx_rows, idx_body, 0)
    plsc.subcore_barrier()

    def fire(chunk, b):
        for g in range(_GPB):
            pltpu.async_copy(
                fused_sh.at[idx_v.at[chunk * _GPB + g]],
                rows_v.at[b].at[pl.ds(g * _IDX_W, _IDX_W)],
                sem_g[b])

    def drain_gather(b):
        pltpu.make_async_copy(out.at[pl.ds(0, _CHUNK)], rows_v.at[b],
                              sem_g[b]).wait()

    def drain_store(b):
        pltpu.make_async_copy(out.at[pl.ds(0, _CHUNK)], rows_v.at[b],
                              sem_s[b]).wait()

    fire(0, 0)

    def chunk_body(i, carry):
        for b in range(2):
            chunk = i * 2 + b
            other = 1 - b

            @pl.when(chunk >= 1)
            def _():
                drain_store(other)

            @pl.when(chunk + 1 < n_chunks)
            def _():
                fire(chunk + 1, other)

            drain_gather(b)
            pltpu.async_copy(rows_v.at[b],
                             out.at[pl.ds(obase + chunk * _CHUNK, _CHUNK)],
                             sem_s[b])
        return carry

    lax.fori_loop(0, n_chunks // 2, chunk_body, 0)
    drain_store((n_chunks - 1) % 2)


def _sc_gather(seq2d, fused):
    info = plsc.get_sparse_core_info()
    nc, ns = info.num_cores, info.num_subcores
    nw = nc * ns
    rows_per_tile = _NROWS // nw
    mesh = plsc.VectorSubcoreMesh(core_axis_name="c", subcore_axis_name="s")
    kern = pl.kernel(
        functools.partial(_sc_gather_body, nc, ns, rows_per_tile),
        out_type=jax.ShapeDtypeStruct((_NROWS, _EMBED), jnp.float32),
        mesh=mesh,
        compiler_params=pltpu.CompilerParams(use_tc_tiling_on_sc=False),
        scratch_types=[
            pltpu.VMEM((rows_per_tile // _IDX_W, _IDX_W), jnp.int32),
            pltpu.VMEM((2, _CHUNK, _EMBED), jnp.float32),
            pltpu.VMEM_SHARED((_FUSED_ROWS, _EMBED), jnp.float32),
            pltpu.SemaphoreType.DMA,
            pltpu.SemaphoreType.DMA,
            pltpu.SemaphoreType.DMA,
            pltpu.SemaphoreType.DMA,
        ],
    )
    return kern(seq2d, fused)


def kernel(seq, seq_table, pos_table):
    seq2d = seq.astype(jnp.int32).reshape(_NROWS // _IDX_W, _IDX_W)
    fused = _build_fused(seq_table.astype(jnp.float32),
                         pos_table.astype(jnp.float32))
    out = _sc_gather(seq2d, fused)
    return out.reshape(_BATCH, _MAX_LEN, _EMBED)
